# baseline (speedup 1.0000x reference)
import functools

import jax
import jax.numpy as jnp
from jax import lax
from jax.experimental import pallas as pl
from jax.experimental.pallas import tpu as pltpu
from jax.experimental.pallas import tpu_sc as plsc

_NUM_CORES = 2
_NUM_SUBCORES = 16
_NUM_WORKERS = _NUM_CORES * _NUM_SUBCORES
_LANES = 16
_CHUNK = 128
_GROUPS = 4


@jax.jit
def _lr_pooled_lookup(xt, table, bias):
    F, NW, G, _one, C = xt.shape
    bpw = G * C
    B = NW * bpw
    mesh = plsc.VectorSubcoreMesh(core_axis_name="c", subcore_axis_name="s")

    @functools.partial(
        pl.kernel,
        out_type=jax.ShapeDtypeStruct((B,), jnp.float32),
        mesh=mesh,
        compiler_params=pltpu.CompilerParams(skip_device_barrier=True),
        scratch_types=[
            pltpu.VMEM((F, G, 1, C), jnp.int32),
            pltpu.VMEM((F, G, 1, C), jnp.float32),
            pltpu.VMEM((_LANES,), jnp.float32),
            pltpu.VMEM((bpw,), jnp.float32),
            [pltpu.SemaphoreType.DMA] * _GROUPS,
            pltpu.SemaphoreType.DMA,
        ],
    )
    def k(xt_hbm, tab_hbm, bias_hbm, out_hbm, xt_v, vals_v, bias_v, acc_v, gsems, osem):
        wid = lax.axis_index("s") * _NUM_CORES + lax.axis_index("c")
        base = wid * bpw
        pltpu.sync_copy(xt_hbm.at[:, wid], xt_v)
        pltpu.sync_copy(bias_hbm, bias_v.at[pl.ds(0, 1)])
        gathers = [
            [
                pltpu.async_copy(
                    tab_hbm.at[xt_v.at[f, g]], vals_v.at[f, g], gsems[g]
                )
                for f in range(F)
            ]
            for g in range(G)
        ]
        bvec = jnp.full((_LANES,), bias_v[...][0], jnp.float32)
        outs = []
        for g in range(G):
            for c in gathers[g]:
                c.wait()
            for i in range(C // _LANES):
                off = i * _LANES
                acc = bvec
                for f in range(F):
                    acc = acc + vals_v[f, g, 0, pl.ds(off, _LANES)]
                acc_v[pl.ds(g * C + off, _LANES)] = acc
            outs.append(
                pltpu.async_copy(
                    acc_v.at[pl.ds(g * C, C)],
                    out_hbm.at[pl.ds(base + g * C, C)],
                    osem,
                )
            )
        for o in outs:
            o.wait()

    return k(xt, table, bias)


def kernel(X, table, bias):
    B, F = X.shape
    bpw = B // _NUM_WORKERS
    G = _GROUPS
    C = bpw // G
    xt = X.T.reshape(F, _NUM_WORKERS, G, 1, C)
    out = _lr_pooled_lookup(xt, table.reshape(1, -1), bias)
    return out.reshape(B, 1)


# speedup vs baseline: 1.0734x; 1.0734x over previous
import functools

import jax
import jax.numpy as jnp
from jax import lax
from jax.experimental import pallas as pl
from jax.experimental.pallas import tpu as pltpu
from jax.experimental.pallas import tpu_sc as plsc

_NUM_CORES = 2
_NUM_SUBCORES = 16
_NUM_WORKERS = _NUM_CORES * _NUM_SUBCORES
_LANES = 16
_CHUNK = 128


@functools.partial(jax.jit, static_argnums=(3,))
def _lr_pooled_lookup(xt, table, bias16, F):
    NW, _one, L = xt.shape
    C = _CHUNK
    bpw = L // F
    NJ = bpw // C
    B = NW * bpw
    mesh = plsc.VectorSubcoreMesh(core_axis_name="c", subcore_axis_name="s")

    @functools.partial(
        pl.kernel,
        out_type=jax.ShapeDtypeStruct((B,), jnp.float32),
        mesh=mesh,
        compiler_params=pltpu.CompilerParams(skip_device_barrier=True),
        scratch_types=[
            pltpu.VMEM((1, 1, L), jnp.int32),
            pltpu.VMEM((1, 1, L), jnp.float32),
            pltpu.VMEM((_LANES,), jnp.float32),
            pltpu.VMEM((bpw,), jnp.float32),
            pltpu.SemaphoreType.DMA,
        ],
    )
    def k(xt_hbm, tab_hbm, bias_hbm, out_hbm, xt_v, vals_v, bias_s, acc_v, gsem):
        wid = lax.axis_index("s") * _NUM_CORES + lax.axis_index("c")
        base = wid * bpw
        pltpu.sync_copy(xt_hbm.at[wid], xt_v.at[0])
        pltpu.sync_copy(bias_hbm, bias_s.at[pl.ds(0, 1)])
        FH = F // 2
        M = FH * bpw
        g1 = pltpu.async_copy(
            tab_hbm.at[xt_v.at[0, :, pl.ds(0, M)]], vals_v.at[0, :, pl.ds(0, M)], gsem
        )
        g2 = pltpu.async_copy(
            tab_hbm.at[xt_v.at[0, :, pl.ds(M, L - M)]],
            vals_v.at[0, :, pl.ds(M, L - M)],
            gsem,
        )
        bvec = jnp.full((_LANES,), bias_s[...][0], jnp.float32)
        per_chunk = C // _LANES
        g1.wait()
        for i in range(bpw // _LANES):
            j, off = i // per_chunk, (i % per_chunk) * _LANES
            acc = bvec
            for f in range(FH):
                acc = acc + vals_v[0, 0, pl.ds((f * NJ + j) * C + off, _LANES)]
            acc_v[pl.ds(i * _LANES, _LANES)] = acc
        g2.wait()
        for i in range(bpw // _LANES):
            j, off = i // per_chunk, (i % per_chunk) * _LANES
            acc = acc_v[pl.ds(i * _LANES, _LANES)]
            for f in range(FH, F):
                acc = acc + vals_v[0, 0, pl.ds((f * NJ + j) * C + off, _LANES)]
            acc_v[pl.ds(i * _LANES, _LANES)] = acc
        pltpu.sync_copy(acc_v, out_hbm.at[pl.ds(base, bpw)])

    return k(xt, table, bias16)


def kernel(X, table, bias):
    B, F = X.shape
    bpw = B // _NUM_WORKERS
    NJ = bpw // _CHUNK
    xt = (
        X.T.reshape(F, _NUM_WORKERS, NJ, _CHUNK)
        .swapaxes(0, 1)
        .reshape(_NUM_WORKERS, 1, F * NJ * _CHUNK)
    )
    out = _lr_pooled_lookup(xt, table.reshape(1, -1), bias, F)
    return out.reshape(B, 1)
